# baseline (speedup 1.0000x reference)
import jax
import jax.numpy as jnp
from jax import lax
from jax.experimental import pallas as pl
from jax.experimental.pallas import tpu as pltpu
from jax.experimental.pallas import tpu_sc as plsc

N = 10000
E = 160000
D_NODE = 128
D_EDGE = 4
D_OUT = 128
D_ATTR = 4

NC = 2
NS = 16
DH = D_OUT // NC
RW = D_EDGE * DH

EPT = E // NS
B = 80
NBT = EPT // B

DRC = 16
NDC = N // DRC
DRT = -(-NDC // NS)

SCALE = 1.0 / 256.0

import numpy as _np
_PERM = _np.empty((D_OUT,), dtype=_np.int32)
for _h in range(NC):
    for _q in range(DH // 32):
        _base = _h * DH + _q * 32
        for _i in range(16):
            _PERM[_base + 2 * _i] = _base + _i
            _PERM[_base + 2 * _i + 1] = _base + 16 + _i


def _prep_body(x_ref, wc_ref, wl_ref, y_ref):
    wl = wl_ref[...]
    x = x_ref[...]
    for k in range(D_EDGE):
        vk = jnp.dot(wc_ref[k], wl, preferred_element_type=jnp.float32) * SCALE
        yk = jnp.dot(x, vk, preferred_element_type=jnp.float32)
        y_ref[:, 0, k * DH:(k + 1) * DH] = yk[:, :DH].astype(jnp.bfloat16)
        y_ref[:, 1, k * DH:(k + 1) * DH] = yk[:, DH:].astype(jnp.bfloat16)


def _tc_prep(node_feats, wc_perm, w_lin1):
    blk = 1000
    return pl.pallas_call(
        _prep_body,
        grid=(N // blk,),
        in_specs=[
            pl.BlockSpec((blk, D_NODE), lambda i: (i, 0)),
            pl.BlockSpec((D_EDGE, D_NODE, D_OUT), lambda i: (0, 0, 0)),
            pl.BlockSpec((D_OUT, D_OUT), lambda i: (0, 0)),
        ],
        out_specs=pl.BlockSpec((blk, NC, RW), lambda i: (i, 0, 0)),
        out_shape=jax.ShapeDtypeStruct((N, NC, RW), jnp.bfloat16),
    )(node_feats, wc_perm, w_lin1)


def _sc_edge_body(y_hbm, ef_hbm, s_hbm, r_hbm, out_hbm,
                  sidx_all, ridx_all, ef2, ridx2, rows2, outv2, zbuf, m_sh,
                  gsem0, gsem1, esem0, esem1, ssem0, ssem1):
    cid = lax.axis_index("c")
    sid = lax.axis_index("s")
    gsems = (gsem0, gsem1)
    esems = (esem0, esem1)
    ssems = (ssem0, ssem1)

    def zrow(i, _):
        for c in range(DH // 16):
            zbuf[i, pl.ds(c * 16, 16)] = jnp.zeros((16,), jnp.float32)
        return 0
    lax.fori_loop(0, DRC, zrow, 0)

    def zcopy(t, _):
        idx = t * NS + sid
        @pl.when(idx < NDC)
        def _():
            pltpu.sync_copy(zbuf, m_sh.at[pl.ds(idx * DRC, DRC)])
        return 0
    lax.fori_loop(0, DRT, zcopy, 0)

    pltpu.sync_copy(s_hbm.at[sid], sidx_all)
    pltpu.sync_copy(r_hbm.at[sid], ridx_all)

    @plsc.parallel_loop(0, EPT // 16, unroll=4)
    def sxf(i):
        v = sidx_all[pl.ds(i * 16, 16)]
        sidx_all[pl.ds(i * 16, 16)] = v * 2 + cid
    del sxf
    plsc.subcore_barrier()

    def start_gather(b, ph):
        pltpu.async_copy(y_hbm.at[sidx_all.at[pl.ds(b * B, B)]],
                         rows2.at[ph], gsems[ph])

    def wait_gather(ph):
        pltpu.make_async_copy(y_hbm.at[sidx_all.at[pl.ds(0, B)]],
                              rows2.at[ph], gsems[ph]).wait()

    def start_ef(b, ph):
        pltpu.async_copy(ef_hbm.at[sid, b], ef2.at[ph], esems[ph])

    def wait_ef(ph):
        pltpu.make_async_copy(ef_hbm.at[sid, 0], ef2.at[ph],
                              esems[ph]).wait()

    def start_scatter(ph):
        pltpu.async_copy(outv2.at[ph], m_sh.at[ridx2.at[ph]],
                         ssems[ph], add=True)

    def wait_scatter(ph):
        pltpu.make_async_copy(outv2.at[ph], m_sh.at[ridx2.at[ph]],
                              ssems[ph]).wait()

    def compute(ph):
        @plsc.parallel_loop(0, B // 4, unroll=2)
        def group(g):
            efq = ef2[ph, pl.ds(g * 16, 16)]
            for j in range(4):
                e = g * 4 + j
                ws = (efq[j * 4], efq[j * 4 + 1], efq[j * 4 + 2],
                      efq[j * 4 + 3])
                for q in range(DH // 32):
                    o32 = q * 32
                    ve = vo = None
                    for k in range(D_EDGE):
                        u = rows2[ph, e, pl.ds(k * DH + o32, 32)]
                        a, bb = plsc.unpack(
                            u, format=plsc.PackFormat.INTERLEAVED,
                            preferred_element_type=jnp.float32)
                        if ve is None:
                            ve = a * ws[k]
                            vo = bb * ws[k]
                        else:
                            ve = ve + a * ws[k]
                            vo = vo + bb * ws[k]
                    outv2[ph, e, pl.ds(o32, 16)] = ve
                    outv2[ph, e, pl.ds(o32 + 16, 16)] = vo
        del group

    start_ef(0, 0)
    start_gather(0, 0)

    def batch(b, _):
        for ph in range(2):
            @pl.when(lax.rem(b, 2) == ph)
            def _():
                @pl.when(b + 1 < NBT)
                def _():
                    start_ef(b + 1, 1 - ph)
                    start_gather(b + 1, 1 - ph)
                wait_gather(ph)
                wait_ef(ph)
                @pl.when(b >= 2)
                def _():
                    wait_scatter(ph)
                for q in range(B // 16):
                    ridx2[ph, pl.ds(q * 16, 16)] = (
                        ridx_all[pl.ds(b * B + q * 16, 16)])
                compute(ph)
                start_scatter(ph)
        return 0
    lax.fori_loop(0, NBT, batch, 0)

    wait_scatter(0)
    wait_scatter(1)
    plsc.subcore_barrier()

    def wcopy(t, _):
        idx = t * NS + sid
        @pl.when(idx < NDC)
        def _():
            pltpu.sync_copy(m_sh.at[pl.ds(idx * DRC, DRC)], zbuf)
            pltpu.sync_copy(zbuf, out_hbm.at[cid, pl.ds(idx * DRC, DRC)])
        return 0
    lax.fori_loop(0, DRT, wcopy, 0)


def _sc_edge(y2, ef3, s4, r3):
    mesh = plsc.VectorSubcoreMesh(core_axis_name="c", subcore_axis_name="s",
                                  num_cores=NC, num_subcores=NS)
    fn = pl.kernel(
        _sc_edge_body,
        out_type=jax.ShapeDtypeStruct((NC, N, DH), jnp.float32),
        mesh=mesh,
        compiler_params=pltpu.CompilerParams(use_tc_tiling_on_sc=False,
                                             needs_layout_passes=False),
        scratch_types=[
            pltpu.VMEM((EPT,), jnp.int32),
            pltpu.VMEM((EPT,), jnp.int32),
            pltpu.VMEM((2, B * 4), jnp.float32),
            pltpu.VMEM((2, B), jnp.int32),
            pltpu.VMEM((2, B, RW), jnp.bfloat16),
            pltpu.VMEM((2, B, DH), jnp.float32),
            pltpu.VMEM((DRC, DH), jnp.float32),
            pltpu.VMEM_SHARED((N, DH), jnp.float32),
            pltpu.SemaphoreType.DMA,
            pltpu.SemaphoreType.DMA,
            pltpu.SemaphoreType.DMA,
            pltpu.SemaphoreType.DMA,
            pltpu.SemaphoreType.DMA,
            pltpu.SemaphoreType.DMA,
        ],
    )
    return fn(y2, ef3, s4, r3)


def _skip_body(mp_ref, attr_ref, ws_ref, wl2_ref, out_ref):
    m = jnp.concatenate([mp_ref[0], mp_ref[1]], axis=1)
    wl2 = wl2_ref[...]
    attrs = attr_ref[...]
    acc = m
    for k in range(D_ATTR):
        vk = jnp.dot(ws_ref[k], wl2, preferred_element_type=jnp.float32) * SCALE
        acc = acc + jnp.dot(m, vk,
                            preferred_element_type=jnp.float32) * attrs[:, k:k + 1]
    out_ref[...] = acc


def _tc_skip(m_parts, node_attrs, ws_perm, w_lin2):
    blk = 1000
    return pl.pallas_call(
        _skip_body,
        grid=(N // blk,),
        in_specs=[
            pl.BlockSpec((NC, blk, DH), lambda i: (0, i, 0)),
            pl.BlockSpec((blk, D_ATTR), lambda i: (i, 0)),
            pl.BlockSpec((D_ATTR, D_OUT, D_OUT), lambda i: (0, 0, 0)),
            pl.BlockSpec((D_OUT, D_OUT), lambda i: (0, 0)),
        ],
        out_specs=pl.BlockSpec((blk, D_OUT), lambda i: (i, 0)),
        out_shape=jax.ShapeDtypeStruct((N, D_OUT), jnp.float32),
    )(m_parts, node_attrs, ws_perm, w_lin2)


@jax.jit
def kernel(node_feats, node_attrs, edge_feats, edge_index,
           W_conv, W_lin1, W_skip, W_lin2):
    senders3 = edge_index[0].reshape(NS, EPT)
    receivers = edge_index[1].reshape(NS, EPT)
    ef3 = edge_feats.reshape(NS, NBT, B * 4)
    wc_perm = W_conv.transpose(1, 0, 2)
    ws_perm = W_skip.transpose(1, 0, 2)
    wl1_perm = W_lin1[:, _PERM]

    yh = _tc_prep(node_feats, wc_perm, wl1_perm)
    y2 = yh.reshape(NC * N, RW)
    m_parts = _sc_edge(y2, ef3, senders3, receivers)
    return _tc_skip(m_parts, node_attrs, ws_perm, W_lin2)


# speedup vs baseline: 1.0180x; 1.0180x over previous
import jax
import jax.numpy as jnp
from jax import lax
from jax.experimental import pallas as pl
from jax.experimental.pallas import tpu as pltpu
from jax.experimental.pallas import tpu_sc as plsc

N = 10000
E = 160000
D_NODE = 128
D_EDGE = 4
D_OUT = 128
D_ATTR = 4

NC = 2
NS = 16
DH = D_OUT // NC
RW = D_EDGE * DH

EPT = E // NS
B = 80
NBT = EPT // B

ZR = 312

SCALE = 1.0 / 256.0

import numpy as _np
_PERM = _np.empty((D_OUT,), dtype=_np.int32)
for _h in range(NC):
    for _q in range(DH // 32):
        _base = _h * DH + _q * 32
        for _i in range(16):
            _PERM[_base + 2 * _i] = _base + _i
            _PERM[_base + 2 * _i + 1] = _base + 16 + _i


def _prep_body(x_ref, wc_ref, wl_ref, y_ref):
    wl = wl_ref[...]
    x = x_ref[...]
    for k in range(D_EDGE):
        vk = jnp.dot(wc_ref[k], wl, preferred_element_type=jnp.float32) * SCALE
        yk = jnp.dot(x, vk, preferred_element_type=jnp.float32)
        y_ref[:, 0, k * DH:(k + 1) * DH] = yk[:, :DH].astype(jnp.bfloat16)
        y_ref[:, 1, k * DH:(k + 1) * DH] = yk[:, DH:].astype(jnp.bfloat16)


def _tc_prep(node_feats, wc_perm, w_lin1):
    blk = 1000
    return pl.pallas_call(
        _prep_body,
        grid=(N // blk,),
        in_specs=[
            pl.BlockSpec((blk, D_NODE), lambda i: (i, 0)),
            pl.BlockSpec((D_EDGE, D_NODE, D_OUT), lambda i: (0, 0, 0)),
            pl.BlockSpec((D_OUT, D_OUT), lambda i: (0, 0)),
        ],
        out_specs=pl.BlockSpec((blk, NC, RW), lambda i: (i, 0, 0)),
        out_shape=jax.ShapeDtypeStruct((N, NC, RW), jnp.bfloat16),
    )(node_feats, wc_perm, w_lin1)


def _sc_edge_body(y_hbm, ef_hbm, s_hbm, r_hbm, out_hbm,
                  sidx_all, ridx_all, ef2, ridx2, rows2, outv2, zbuf, m_sh,
                  gsem0, gsem1, esem0, esem1, ssem0, ssem1):
    cid = lax.axis_index("c")
    sid = lax.axis_index("s")
    gsems = (gsem0, gsem1)
    esems = (esem0, esem1)
    ssems = (ssem0, ssem1)

    @plsc.parallel_loop(0, ZR, unroll=4)
    def zrow(i):
        for c in range(DH // 16):
            zbuf[i, pl.ds(c * 16, 16)] = jnp.zeros((16,), jnp.float32)
    del zrow

    row0 = sid * 624
    pltpu.sync_copy(zbuf, m_sh.at[pl.ds(row0, ZR)])
    pltpu.sync_copy(zbuf, m_sh.at[pl.ds(row0 + ZR, ZR)])
    @pl.when(sid == NS - 1)
    def _():
        pltpu.sync_copy(zbuf.at[pl.ds(0, 16)], m_sh.at[pl.ds(9984, 16)])

    pltpu.sync_copy(s_hbm.at[sid], sidx_all)
    pltpu.sync_copy(r_hbm.at[sid], ridx_all)

    @plsc.parallel_loop(0, EPT // 16, unroll=4)
    def sxf(i):
        v = sidx_all[pl.ds(i * 16, 16)]
        sidx_all[pl.ds(i * 16, 16)] = v * 2 + cid
    del sxf
    plsc.subcore_barrier()

    def start_gather(b, ph):
        pltpu.async_copy(y_hbm.at[sidx_all.at[pl.ds(b * B, B)]],
                         rows2.at[ph], gsems[ph])

    def wait_gather(ph):
        pltpu.make_async_copy(y_hbm.at[sidx_all.at[pl.ds(0, B)]],
                              rows2.at[ph], gsems[ph]).wait()

    def start_ef(b, ph):
        pltpu.async_copy(ef_hbm.at[sid, b], ef2.at[ph], esems[ph])

    def wait_ef(ph):
        pltpu.make_async_copy(ef_hbm.at[sid, 0], ef2.at[ph],
                              esems[ph]).wait()

    def start_scatter(ph):
        pltpu.async_copy(outv2.at[ph], m_sh.at[ridx2.at[ph]],
                         ssems[ph], add=True)

    def wait_scatter(ph):
        pltpu.make_async_copy(outv2.at[ph], m_sh.at[ridx2.at[ph]],
                              ssems[ph]).wait()

    def compute(ph):
        @plsc.parallel_loop(0, B // 4, unroll=2)
        def group(g):
            efq = ef2[ph, pl.ds(g * 16, 16)]
            for j in range(4):
                e = g * 4 + j
                ws = (efq[j * 4], efq[j * 4 + 1], efq[j * 4 + 2],
                      efq[j * 4 + 3])
                for q in range(DH // 32):
                    o32 = q * 32
                    ve = vo = None
                    for k in range(D_EDGE):
                        u = rows2[ph, e, pl.ds(k * DH + o32, 32)]
                        a, bb = plsc.unpack(
                            u, format=plsc.PackFormat.INTERLEAVED,
                            preferred_element_type=jnp.float32)
                        if ve is None:
                            ve = a * ws[k]
                            vo = bb * ws[k]
                        else:
                            ve = ve + a * ws[k]
                            vo = vo + bb * ws[k]
                    outv2[ph, e, pl.ds(o32, 16)] = ve
                    outv2[ph, e, pl.ds(o32 + 16, 16)] = vo
        del group

    start_ef(0, 0)
    start_gather(0, 0)

    def batch(b, _):
        for ph in range(2):
            @pl.when(lax.rem(b, 2) == ph)
            def _():
                @pl.when(b + 1 < NBT)
                def _():
                    start_ef(b + 1, 1 - ph)
                    start_gather(b + 1, 1 - ph)
                wait_gather(ph)
                wait_ef(ph)
                @pl.when(b >= 2)
                def _():
                    wait_scatter(ph)
                for q in range(B // 16):
                    ridx2[ph, pl.ds(q * 16, 16)] = (
                        ridx_all[pl.ds(b * B + q * 16, 16)])
                compute(ph)
                start_scatter(ph)
        return 0
    lax.fori_loop(0, NBT, batch, 0)

    wait_scatter(0)
    wait_scatter(1)
    plsc.subcore_barrier()

    pltpu.sync_copy(m_sh.at[pl.ds(row0, ZR)],
                    out_hbm.at[cid, pl.ds(row0, ZR)])
    pltpu.sync_copy(m_sh.at[pl.ds(row0 + ZR, ZR)],
                    out_hbm.at[cid, pl.ds(row0 + ZR, ZR)])
    @pl.when(sid == NS - 1)
    def _():
        pltpu.sync_copy(m_sh.at[pl.ds(9984, 16)],
                        out_hbm.at[cid, pl.ds(9984, 16)])


def _sc_edge(y2, ef3, s4, r3):
    mesh = plsc.VectorSubcoreMesh(core_axis_name="c", subcore_axis_name="s",
                                  num_cores=NC, num_subcores=NS)
    fn = pl.kernel(
        _sc_edge_body,
        out_type=jax.ShapeDtypeStruct((NC, N, DH), jnp.float32),
        mesh=mesh,
        compiler_params=pltpu.CompilerParams(use_tc_tiling_on_sc=False,
                                             needs_layout_passes=False),
        scratch_types=[
            pltpu.VMEM((EPT,), jnp.int32),
            pltpu.VMEM((EPT,), jnp.int32),
            pltpu.VMEM((2, B * 4), jnp.float32),
            pltpu.VMEM((2, B), jnp.int32),
            pltpu.VMEM((2, B, RW), jnp.bfloat16),
            pltpu.VMEM((2, B, DH), jnp.float32),
            pltpu.VMEM((ZR, DH), jnp.float32),
            pltpu.VMEM_SHARED((N, DH), jnp.float32),
            pltpu.SemaphoreType.DMA,
            pltpu.SemaphoreType.DMA,
            pltpu.SemaphoreType.DMA,
            pltpu.SemaphoreType.DMA,
            pltpu.SemaphoreType.DMA,
            pltpu.SemaphoreType.DMA,
        ],
    )
    return fn(y2, ef3, s4, r3)


def _skip_body(mp_ref, attr_ref, ws_ref, wl2_ref, out_ref):
    m = jnp.concatenate([mp_ref[0], mp_ref[1]], axis=1)
    wl2 = wl2_ref[...]
    attrs = attr_ref[...]
    acc = m
    for k in range(D_ATTR):
        vk = jnp.dot(ws_ref[k], wl2, preferred_element_type=jnp.float32) * SCALE
        acc = acc + jnp.dot(m, vk,
                            preferred_element_type=jnp.float32) * attrs[:, k:k + 1]
    out_ref[...] = acc


def _tc_skip(m_parts, node_attrs, ws_perm, w_lin2):
    blk = 1000
    return pl.pallas_call(
        _skip_body,
        grid=(N // blk,),
        in_specs=[
            pl.BlockSpec((NC, blk, DH), lambda i: (0, i, 0)),
            pl.BlockSpec((blk, D_ATTR), lambda i: (i, 0)),
            pl.BlockSpec((D_ATTR, D_OUT, D_OUT), lambda i: (0, 0, 0)),
            pl.BlockSpec((D_OUT, D_OUT), lambda i: (0, 0)),
        ],
        out_specs=pl.BlockSpec((blk, D_OUT), lambda i: (i, 0)),
        out_shape=jax.ShapeDtypeStruct((N, D_OUT), jnp.float32),
    )(m_parts, node_attrs, ws_perm, w_lin2)


@jax.jit
def kernel(node_feats, node_attrs, edge_feats, edge_index,
           W_conv, W_lin1, W_skip, W_lin2):
    senders3 = edge_index[0].reshape(NS, EPT)
    receivers = edge_index[1].reshape(NS, EPT)
    ef3 = edge_feats.reshape(NS, NBT, B * 4)
    wc_perm = W_conv.transpose(1, 0, 2)
    ws_perm = W_skip.transpose(1, 0, 2)
    wl1_perm = W_lin1[:, _PERM]

    yh = _tc_prep(node_feats, wc_perm, wl1_perm)
    y2 = yh.reshape(NC * N, RW)
    m_parts = _sc_edge(y2, ef3, senders3, receivers)
    return _tc_skip(m_parts, node_attrs, ws_perm, W_lin2)


# speedup vs baseline: 1.1639x; 1.1433x over previous
import jax
import jax.numpy as jnp
from jax import lax
from jax.experimental import pallas as pl
from jax.experimental.pallas import tpu as pltpu
from jax.experimental.pallas import tpu_sc as plsc

N = 10000
E = 160000
D_NODE = 128
D_EDGE = 4
D_OUT = 128
D_ATTR = 4

NC = 2
NS = 16
DH = D_OUT // NC
RW = D_EDGE * DH

EPT = E // NS
B = 80
NBT = EPT // B

ZR = 312

SCALE = 1.0 / 256.0

import numpy as _np
_PERM = _np.empty((D_OUT,), dtype=_np.int32)
for _h in range(NC):
    for _q in range(DH // 32):
        _base = _h * DH + _q * 32
        for _i in range(16):
            _PERM[_base + 2 * _i] = _base + _i
            _PERM[_base + 2 * _i + 1] = _base + 16 + _i


def _prep_body(x_ref, wc_ref, wl_ref, y_ref):
    wl = wl_ref[...]
    x = x_ref[...]
    vs = [jnp.dot(wc_ref[k], wl, preferred_element_type=jnp.float32) * SCALE
          for k in range(D_EDGE)]
    vfull = jnp.concatenate(
        [vs[k][:, h * DH:(h + 1) * DH] for h in range(NC)
         for k in range(D_EDGE)], axis=1)
    y = jnp.dot(x, vfull, preferred_element_type=jnp.float32)
    y_ref[...] = y.astype(jnp.bfloat16)


def _tc_prep(node_feats, wc_perm, w_lin1):
    blk = 1000
    return pl.pallas_call(
        _prep_body,
        grid=(N // blk,),
        in_specs=[
            pl.BlockSpec((blk, D_NODE), lambda i: (i, 0)),
            pl.BlockSpec((D_EDGE, D_NODE, D_OUT), lambda i: (0, 0, 0)),
            pl.BlockSpec((D_OUT, D_OUT), lambda i: (0, 0)),
        ],
        out_specs=pl.BlockSpec((blk, NC * RW), lambda i: (i, 0)),
        out_shape=jax.ShapeDtypeStruct((N, NC * RW), jnp.bfloat16),
    )(node_feats, wc_perm, w_lin1)


def _sc_edge_body(y_hbm, ef_hbm, s_hbm, r_hbm, out_hbm,
                  sidx_all, ridx_all, ef2, ridx2, rows2, outv2, zbuf, m_sh,
                  gsem0, gsem1, esem0, esem1, ssem0, ssem1):
    cid = lax.axis_index("c")
    sid = lax.axis_index("s")
    gsems = (gsem0, gsem1)
    esems = (esem0, esem1)
    ssems = (ssem0, ssem1)

    @plsc.parallel_loop(0, ZR, unroll=4)
    def zrow(i):
        for c in range(DH // 16):
            zbuf[i, pl.ds(c * 16, 16)] = jnp.zeros((16,), jnp.float32)
    del zrow

    row0 = sid * 624
    pltpu.sync_copy(zbuf, m_sh.at[pl.ds(row0, ZR)])
    pltpu.sync_copy(zbuf, m_sh.at[pl.ds(row0 + ZR, ZR)])
    @pl.when(sid == NS - 1)
    def _():
        pltpu.sync_copy(zbuf.at[pl.ds(0, 16)], m_sh.at[pl.ds(9984, 16)])

    pltpu.sync_copy(s_hbm.at[sid], sidx_all)
    pltpu.sync_copy(r_hbm.at[sid], ridx_all)

    @plsc.parallel_loop(0, EPT // 16, unroll=4)
    def sxf(i):
        v = sidx_all[pl.ds(i * 16, 16)]
        sidx_all[pl.ds(i * 16, 16)] = v * 2 + cid
    del sxf
    plsc.subcore_barrier()

    def start_gather(b, ph):
        pltpu.async_copy(y_hbm.at[sidx_all.at[pl.ds(b * B, B)]],
                         rows2.at[ph], gsems[ph])

    def wait_gather(ph):
        pltpu.make_async_copy(y_hbm.at[sidx_all.at[pl.ds(0, B)]],
                              rows2.at[ph], gsems[ph]).wait()

    def start_ef(b, ph):
        pltpu.async_copy(ef_hbm.at[sid, b], ef2.at[ph], esems[ph])

    def wait_ef(ph):
        pltpu.make_async_copy(ef_hbm.at[sid, 0], ef2.at[ph],
                              esems[ph]).wait()

    def start_scatter(ph):
        pltpu.async_copy(outv2.at[ph], m_sh.at[ridx2.at[ph]],
                         ssems[ph], add=True)

    def wait_scatter(ph):
        pltpu.make_async_copy(outv2.at[ph], m_sh.at[ridx2.at[ph]],
                              ssems[ph]).wait()

    def compute(ph):
        @plsc.parallel_loop(0, B // 4, unroll=2)
        def group(g):
            efq = ef2[ph, pl.ds(g * 16, 16)]
            for j in range(4):
                e = g * 4 + j
                ws = (efq[j * 4], efq[j * 4 + 1], efq[j * 4 + 2],
                      efq[j * 4 + 3])
                for q in range(DH // 32):
                    o32 = q * 32
                    ve = vo = None
                    for k in range(D_EDGE):
                        u = rows2[ph, e, pl.ds(k * DH + o32, 32)]
                        a, bb = plsc.unpack(
                            u, format=plsc.PackFormat.INTERLEAVED,
                            preferred_element_type=jnp.float32)
                        if ve is None:
                            ve = a * ws[k]
                            vo = bb * ws[k]
                        else:
                            ve = ve + a * ws[k]
                            vo = vo + bb * ws[k]
                    outv2[ph, e, pl.ds(o32, 16)] = ve
                    outv2[ph, e, pl.ds(o32 + 16, 16)] = vo
        del group

    start_ef(0, 0)
    start_gather(0, 0)

    def batch(b, _):
        for ph in range(2):
            @pl.when(lax.rem(b, 2) == ph)
            def _():
                @pl.when(b + 1 < NBT)
                def _():
                    start_ef(b + 1, 1 - ph)
                    start_gather(b + 1, 1 - ph)
                wait_gather(ph)
                wait_ef(ph)
                @pl.when(b >= 2)
                def _():
                    wait_scatter(ph)
                for q in range(B // 16):
                    ridx2[ph, pl.ds(q * 16, 16)] = (
                        ridx_all[pl.ds(b * B + q * 16, 16)])
                compute(ph)
                start_scatter(ph)
        return 0
    lax.fori_loop(0, NBT, batch, 0)

    wait_scatter(0)
    wait_scatter(1)
    plsc.subcore_barrier()

    pltpu.sync_copy(m_sh.at[pl.ds(row0, ZR)],
                    out_hbm.at[cid, pl.ds(row0, ZR)])
    pltpu.sync_copy(m_sh.at[pl.ds(row0 + ZR, ZR)],
                    out_hbm.at[cid, pl.ds(row0 + ZR, ZR)])
    @pl.when(sid == NS - 1)
    def _():
        pltpu.sync_copy(m_sh.at[pl.ds(9984, 16)],
                        out_hbm.at[cid, pl.ds(9984, 16)])


def _sc_edge(y2, ef3, s4, r3):
    mesh = plsc.VectorSubcoreMesh(core_axis_name="c", subcore_axis_name="s",
                                  num_cores=NC, num_subcores=NS)
    fn = pl.kernel(
        _sc_edge_body,
        out_type=jax.ShapeDtypeStruct((NC, N, DH), jnp.float32),
        mesh=mesh,
        compiler_params=pltpu.CompilerParams(use_tc_tiling_on_sc=False,
                                             needs_layout_passes=False),
        scratch_types=[
            pltpu.VMEM((EPT,), jnp.int32),
            pltpu.VMEM((EPT,), jnp.int32),
            pltpu.VMEM((2, B * 4), jnp.float32),
            pltpu.VMEM((2, B), jnp.int32),
            pltpu.VMEM((2, B, RW), jnp.bfloat16),
            pltpu.VMEM((2, B, DH), jnp.float32),
            pltpu.VMEM((ZR, DH), jnp.float32),
            pltpu.VMEM_SHARED((N, DH), jnp.float32),
            pltpu.SemaphoreType.DMA,
            pltpu.SemaphoreType.DMA,
            pltpu.SemaphoreType.DMA,
            pltpu.SemaphoreType.DMA,
            pltpu.SemaphoreType.DMA,
            pltpu.SemaphoreType.DMA,
        ],
    )
    return fn(y2, ef3, s4, r3)


def _skip_body(mp_ref, attr_ref, ws_ref, wl2_ref, out_ref):
    m = jnp.concatenate([mp_ref[0], mp_ref[1]], axis=1)
    wl2 = wl2_ref[...]
    attrs = attr_ref[...]
    acc = m
    for k in range(D_ATTR):
        vk = jnp.dot(ws_ref[k], wl2, preferred_element_type=jnp.float32) * SCALE
        acc = acc + jnp.dot(m, vk,
                            preferred_element_type=jnp.float32) * attrs[:, k:k + 1]
    out_ref[...] = acc


def _tc_skip(m_parts, node_attrs, ws_perm, w_lin2):
    blk = 1000
    return pl.pallas_call(
        _skip_body,
        grid=(N // blk,),
        in_specs=[
            pl.BlockSpec((NC, blk, DH), lambda i: (0, i, 0)),
            pl.BlockSpec((blk, D_ATTR), lambda i: (i, 0)),
            pl.BlockSpec((D_ATTR, D_OUT, D_OUT), lambda i: (0, 0, 0)),
            pl.BlockSpec((D_OUT, D_OUT), lambda i: (0, 0)),
        ],
        out_specs=pl.BlockSpec((blk, D_OUT), lambda i: (i, 0)),
        out_shape=jax.ShapeDtypeStruct((N, D_OUT), jnp.float32),
    )(m_parts, node_attrs, ws_perm, w_lin2)


@jax.jit
def kernel(node_feats, node_attrs, edge_feats, edge_index,
           W_conv, W_lin1, W_skip, W_lin2):
    senders3 = edge_index[0].reshape(NS, EPT)
    receivers = edge_index[1].reshape(NS, EPT)
    ef3 = edge_feats.reshape(NS, NBT, B * 4)
    wc_perm = W_conv.transpose(1, 0, 2)
    ws_perm = W_skip.transpose(1, 0, 2)
    wl1_perm = W_lin1[:, _PERM]

    yh = _tc_prep(node_feats, wc_perm, wl1_perm)
    y2 = yh.reshape(NC * N, RW)
    m_parts = _sc_edge(y2, ef3, senders3, receivers)
    return _tc_skip(m_parts, node_attrs, ws_perm, W_lin2)


# speedup vs baseline: 1.2285x; 1.0555x over previous
import jax
import jax.numpy as jnp
from jax import lax
from jax.experimental import pallas as pl
from jax.experimental.pallas import tpu as pltpu
from jax.experimental.pallas import tpu_sc as plsc

N = 10000
E = 160000
D_NODE = 128
D_EDGE = 4
D_OUT = 128
D_ATTR = 4

NC = 2
NS = 16
DH = D_OUT // NC
RW = D_EDGE * DH

EPT = E // NS
B = 80
NBT = EPT // B

ZR = 312

SCALE = 1.0 / 256.0

import numpy as _np
_PERM = _np.empty((D_OUT,), dtype=_np.int32)
for _h in range(NC):
    for _q in range(DH // 32):
        _base = _h * DH + _q * 32
        for _i in range(16):
            _PERM[_base + 2 * _i] = _base + _i
            _PERM[_base + 2 * _i + 1] = _base + 16 + _i


def _prep_body(x_ref, wc_ref, wl_ref, y_ref):
    wl = wl_ref[...]
    x = x_ref[...]
    vs = [jnp.dot(wc_ref[k], wl, preferred_element_type=jnp.float32) * SCALE
          for k in range(D_EDGE)]
    vfull = jnp.concatenate(
        [vs[k][:, h * DH:(h + 1) * DH] for h in range(NC)
         for k in range(D_EDGE)], axis=1)
    y = jnp.dot(x, vfull, preferred_element_type=jnp.float32)
    yb = y.astype(jnp.bfloat16)
    y_ref[0] = yb[:, :RW]
    y_ref[1] = yb[:, RW:]


def _tc_prep(node_feats, wc_perm, w_lin1):
    blk = 1000
    return pl.pallas_call(
        _prep_body,
        grid=(N // blk,),
        in_specs=[
            pl.BlockSpec((blk, D_NODE), lambda i: (i, 0)),
            pl.BlockSpec((D_EDGE, D_NODE, D_OUT), lambda i: (0, 0, 0)),
            pl.BlockSpec((D_OUT, D_OUT), lambda i: (0, 0)),
        ],
        out_specs=pl.BlockSpec((NC, blk, RW), lambda i: (0, i, 0)),
        out_shape=jax.ShapeDtypeStruct((NC, N, RW), jnp.bfloat16),
    )(node_feats, wc_perm, w_lin1)


def _sc_edge_body(y_hbm, ef_hbm, s_hbm, r_hbm, out_hbm,
                  sidx_all, ridx_all, ef2, ridx2, rows2, outv2, zbuf, m_sh,
                  gsem0, gsem1, esem0, esem1, ssem0, ssem1):
    cid = lax.axis_index("c")
    sid = lax.axis_index("s")
    gsems = (gsem0, gsem1)
    esems = (esem0, esem1)
    ssems = (ssem0, ssem1)

    @plsc.parallel_loop(0, ZR, unroll=4)
    def zrow(i):
        for c in range(DH // 16):
            zbuf[i, pl.ds(c * 16, 16)] = jnp.zeros((16,), jnp.float32)
    del zrow

    row0 = sid * 624
    pltpu.sync_copy(zbuf, m_sh.at[pl.ds(row0, ZR)])
    pltpu.sync_copy(zbuf, m_sh.at[pl.ds(row0 + ZR, ZR)])
    @pl.when(sid == NS - 1)
    def _():
        pltpu.sync_copy(zbuf.at[pl.ds(0, 16)], m_sh.at[pl.ds(9984, 16)])

    pltpu.sync_copy(s_hbm.at[sid], sidx_all)
    pltpu.sync_copy(r_hbm.at[sid], ridx_all)

    @plsc.parallel_loop(0, EPT // 16, unroll=4)
    def sxf(i):
        v = sidx_all[pl.ds(i * 16, 16)]
        sidx_all[pl.ds(i * 16, 16)] = v + cid * N
    del sxf
    plsc.subcore_barrier()

    def start_gather(b, ph):
        pltpu.async_copy(y_hbm.at[sidx_all.at[pl.ds(b * B, B)]],
                         rows2.at[ph], gsems[ph])

    def wait_gather(ph):
        pltpu.make_async_copy(y_hbm.at[sidx_all.at[pl.ds(0, B)]],
                              rows2.at[ph], gsems[ph]).wait()

    def start_ef(b, ph):
        pltpu.async_copy(ef_hbm.at[sid, b], ef2.at[ph], esems[ph])

    def wait_ef(ph):
        pltpu.make_async_copy(ef_hbm.at[sid, 0], ef2.at[ph],
                              esems[ph]).wait()

    def start_scatter(ph):
        pltpu.async_copy(outv2.at[ph], m_sh.at[ridx2.at[ph]],
                         ssems[ph], add=True)

    def wait_scatter(ph):
        pltpu.make_async_copy(outv2.at[ph], m_sh.at[ridx2.at[ph]],
                              ssems[ph]).wait()

    def compute(ph):
        @plsc.parallel_loop(0, B // 4, unroll=2)
        def group(g):
            efq = ef2[ph, pl.ds(g * 16, 16)]
            for j in range(4):
                e = g * 4 + j
                ws = (efq[j * 4], efq[j * 4 + 1], efq[j * 4 + 2],
                      efq[j * 4 + 3])
                for q in range(DH // 32):
                    o32 = q * 32
                    ve = vo = None
                    for k in range(D_EDGE):
                        u = rows2[ph, e, pl.ds(k * DH + o32, 32)]
                        a, bb = plsc.unpack(
                            u, format=plsc.PackFormat.INTERLEAVED,
                            preferred_element_type=jnp.float32)
                        if ve is None:
                            ve = a * ws[k]
                            vo = bb * ws[k]
                        else:
                            ve = ve + a * ws[k]
                            vo = vo + bb * ws[k]
                    outv2[ph, e, pl.ds(o32, 16)] = ve
                    outv2[ph, e, pl.ds(o32 + 16, 16)] = vo
        del group

    start_ef(0, 0)
    start_gather(0, 0)

    def batch(b, _):
        for ph in range(2):
            @pl.when(lax.rem(b, 2) == ph)
            def _():
                @pl.when(b + 1 < NBT)
                def _():
                    start_ef(b + 1, 1 - ph)
                    start_gather(b + 1, 1 - ph)
                wait_gather(ph)
                wait_ef(ph)
                @pl.when(b >= 2)
                def _():
                    wait_scatter(ph)
                for q in range(B // 16):
                    ridx2[ph, pl.ds(q * 16, 16)] = (
                        ridx_all[pl.ds(b * B + q * 16, 16)])
                compute(ph)
                start_scatter(ph)
        return 0
    lax.fori_loop(0, NBT, batch, 0)

    wait_scatter(0)
    wait_scatter(1)
    plsc.subcore_barrier()

    pltpu.sync_copy(m_sh.at[pl.ds(row0, ZR)],
                    out_hbm.at[cid, pl.ds(row0, ZR)])
    pltpu.sync_copy(m_sh.at[pl.ds(row0 + ZR, ZR)],
                    out_hbm.at[cid, pl.ds(row0 + ZR, ZR)])
    @pl.when(sid == NS - 1)
    def _():
        pltpu.sync_copy(m_sh.at[pl.ds(9984, 16)],
                        out_hbm.at[cid, pl.ds(9984, 16)])


def _sc_edge(y2, ef3, s4, r3):
    mesh = plsc.VectorSubcoreMesh(core_axis_name="c", subcore_axis_name="s",
                                  num_cores=NC, num_subcores=NS)
    fn = pl.kernel(
        _sc_edge_body,
        out_type=jax.ShapeDtypeStruct((NC, N, DH), jnp.float32),
        mesh=mesh,
        compiler_params=pltpu.CompilerParams(use_tc_tiling_on_sc=False,
                                             needs_layout_passes=False),
        scratch_types=[
            pltpu.VMEM((EPT,), jnp.int32),
            pltpu.VMEM((EPT,), jnp.int32),
            pltpu.VMEM((2, B * 4), jnp.float32),
            pltpu.VMEM((2, B), jnp.int32),
            pltpu.VMEM((2, B, RW), jnp.bfloat16),
            pltpu.VMEM((2, B, DH), jnp.float32),
            pltpu.VMEM((ZR, DH), jnp.float32),
            pltpu.VMEM_SHARED((N, DH), jnp.float32),
            pltpu.SemaphoreType.DMA,
            pltpu.SemaphoreType.DMA,
            pltpu.SemaphoreType.DMA,
            pltpu.SemaphoreType.DMA,
            pltpu.SemaphoreType.DMA,
            pltpu.SemaphoreType.DMA,
        ],
    )
    return fn(y2, ef3, s4, r3)


def _skip_body(mp_ref, attr_ref, ws_ref, wl2_ref, out_ref):
    m = jnp.concatenate([mp_ref[0], mp_ref[1]], axis=1)
    wl2 = wl2_ref[...]
    attrs = attr_ref[...]
    acc = m
    for k in range(D_ATTR):
        vk = jnp.dot(ws_ref[k], wl2, preferred_element_type=jnp.float32) * SCALE
        acc = acc + jnp.dot(m, vk,
                            preferred_element_type=jnp.float32) * attrs[:, k:k + 1]
    out_ref[...] = acc


def _tc_skip(m_parts, node_attrs, ws_perm, w_lin2):
    blk = 1000
    return pl.pallas_call(
        _skip_body,
        grid=(N // blk,),
        in_specs=[
            pl.BlockSpec((NC, blk, DH), lambda i: (0, i, 0)),
            pl.BlockSpec((blk, D_ATTR), lambda i: (i, 0)),
            pl.BlockSpec((D_ATTR, D_OUT, D_OUT), lambda i: (0, 0, 0)),
            pl.BlockSpec((D_OUT, D_OUT), lambda i: (0, 0)),
        ],
        out_specs=pl.BlockSpec((blk, D_OUT), lambda i: (i, 0)),
        out_shape=jax.ShapeDtypeStruct((N, D_OUT), jnp.float32),
    )(m_parts, node_attrs, ws_perm, w_lin2)


@jax.jit
def kernel(node_feats, node_attrs, edge_feats, edge_index,
           W_conv, W_lin1, W_skip, W_lin2):
    senders3 = edge_index[0].reshape(NS, EPT)
    receivers = edge_index[1].reshape(NS, EPT)
    ef3 = edge_feats.reshape(NS, NBT, B * 4)
    wc_perm = W_conv.transpose(1, 0, 2)
    ws_perm = W_skip.transpose(1, 0, 2)
    wl1_perm = W_lin1[:, _PERM]

    yh = _tc_prep(node_feats, wc_perm, wl1_perm)
    y2 = yh.reshape(NC * N, RW)
    m_parts = _sc_edge(y2, ef3, senders3, receivers)
    return _tc_skip(m_parts, node_attrs, ws_perm, W_lin2)


# speedup vs baseline: 1.2291x; 1.0004x over previous
import jax
import jax.numpy as jnp
from jax import lax
from jax.experimental import pallas as pl
from jax.experimental.pallas import tpu as pltpu
from jax.experimental.pallas import tpu_sc as plsc

N = 10000
E = 160000
D_NODE = 128
D_EDGE = 4
D_OUT = 128
D_ATTR = 4

NC = 2
NS = 16
DH = D_OUT // NC
RW = D_EDGE * DH

EPT = E // NS
B = 80
NBT = EPT // B

ZR = 312

SCALE = 1.0 / 256.0

import numpy as _np
_PERM = _np.empty((D_OUT,), dtype=_np.int32)
for _h in range(NC):
    for _q in range(DH // 32):
        _base = _h * DH + _q * 32
        for _i in range(16):
            _PERM[_base + 2 * _i] = _base + _i
            _PERM[_base + 2 * _i + 1] = _base + 16 + _i


def _prep_body(x_ref, wc_ref, wl_ref, y_ref):
    wl = wl_ref[...]
    x = x_ref[...]
    vs = [jnp.dot(wc_ref[k], wl, preferred_element_type=jnp.float32) * SCALE
          for k in range(D_EDGE)]
    vfull = jnp.concatenate(
        [vs[k][:, h * DH:(h + 1) * DH] for h in range(NC)
         for k in range(D_EDGE)], axis=1)
    y = jnp.dot(x, vfull, preferred_element_type=jnp.float32)
    yb = y.astype(jnp.bfloat16)
    y_ref[0] = yb[:, :RW]
    y_ref[1] = yb[:, RW:]


def _tc_prep(node_feats, wc_perm, w_lin1):
    blk = 1000
    return pl.pallas_call(
        _prep_body,
        grid=(N // blk,),
        in_specs=[
            pl.BlockSpec((blk, D_NODE), lambda i: (i, 0)),
            pl.BlockSpec((D_EDGE, D_NODE, D_OUT), lambda i: (0, 0, 0)),
            pl.BlockSpec((D_OUT, D_OUT), lambda i: (0, 0)),
        ],
        out_specs=pl.BlockSpec((NC, blk, RW), lambda i: (0, i, 0)),
        out_shape=jax.ShapeDtypeStruct((NC, N, RW), jnp.bfloat16),
    )(node_feats, wc_perm, w_lin1)


def _sc_edge_body(y_hbm, ef_hbm, s_hbm, r_hbm, out_hbm,
                  sidx_all, ridx_all, ef2, ridx2, rows2, outv2, zbuf, m_sh,
                  gsem0, gsem1, esem0, esem1, ssem0, ssem1):
    cid = lax.axis_index("c")
    sid = lax.axis_index("s")
    gsems = (gsem0, gsem1)
    esems = (esem0, esem1)
    ssems = (ssem0, ssem1)

    @plsc.parallel_loop(0, ZR, unroll=4)
    def zrow(i):
        for c in range(DH // 16):
            zbuf[i, pl.ds(c * 16, 16)] = jnp.zeros((16,), jnp.float32)
    del zrow

    row0 = sid * 624
    pltpu.sync_copy(zbuf, m_sh.at[pl.ds(row0, ZR)])
    pltpu.sync_copy(zbuf, m_sh.at[pl.ds(row0 + ZR, ZR)])
    @pl.when(sid == NS - 1)
    def _():
        pltpu.sync_copy(zbuf.at[pl.ds(0, 16)], m_sh.at[pl.ds(9984, 16)])

    pltpu.sync_copy(s_hbm.at[pl.ds(sid * EPT, EPT)], sidx_all)
    pltpu.sync_copy(r_hbm.at[pl.ds(sid * EPT, EPT)], ridx_all)

    @plsc.parallel_loop(0, EPT // 16, unroll=4)
    def sxf(i):
        v = sidx_all[pl.ds(i * 16, 16)]
        sidx_all[pl.ds(i * 16, 16)] = v + cid * N
    del sxf
    plsc.subcore_barrier()

    def start_gather(b, ph):
        pltpu.async_copy(y_hbm.at[sidx_all.at[pl.ds(b * B, B)]],
                         rows2.at[ph], gsems[ph])

    def wait_gather(ph):
        pltpu.make_async_copy(y_hbm.at[sidx_all.at[pl.ds(0, B)]],
                              rows2.at[ph], gsems[ph]).wait()

    def start_ef(b, ph):
        pltpu.async_copy(ef_hbm.at[pl.ds(sid * EPT * 4 + b * (B * 4), B * 4)],
                         ef2.at[ph], esems[ph])

    def wait_ef(ph):
        pltpu.make_async_copy(ef_hbm.at[pl.ds(0, B * 4)], ef2.at[ph],
                              esems[ph]).wait()

    def start_scatter(ph):
        pltpu.async_copy(outv2.at[ph], m_sh.at[ridx2.at[ph]],
                         ssems[ph], add=True)

    def wait_scatter(ph):
        pltpu.make_async_copy(outv2.at[ph], m_sh.at[ridx2.at[ph]],
                              ssems[ph]).wait()

    def compute(ph):
        @plsc.parallel_loop(0, B // 4, unroll=2)
        def group(g):
            efq = ef2[ph, pl.ds(g * 16, 16)]
            for j in range(4):
                e = g * 4 + j
                ws = (efq[j * 4], efq[j * 4 + 1], efq[j * 4 + 2],
                      efq[j * 4 + 3])
                for q in range(DH // 32):
                    o32 = q * 32
                    ve = vo = None
                    for k in range(D_EDGE):
                        u = rows2[ph, e, pl.ds(k * DH + o32, 32)]
                        a, bb = plsc.unpack(
                            u, format=plsc.PackFormat.INTERLEAVED,
                            preferred_element_type=jnp.float32)
                        if ve is None:
                            ve = a * ws[k]
                            vo = bb * ws[k]
                        else:
                            ve = ve + a * ws[k]
                            vo = vo + bb * ws[k]
                    outv2[ph, e, pl.ds(o32, 16)] = ve
                    outv2[ph, e, pl.ds(o32 + 16, 16)] = vo
        del group

    start_ef(0, 0)
    start_gather(0, 0)

    def batch(b, _):
        for ph in range(2):
            @pl.when(lax.rem(b, 2) == ph)
            def _():
                @pl.when(b + 1 < NBT)
                def _():
                    start_ef(b + 1, 1 - ph)
                    start_gather(b + 1, 1 - ph)
                wait_gather(ph)
                wait_ef(ph)
                @pl.when(b >= 2)
                def _():
                    wait_scatter(ph)
                for q in range(B // 16):
                    ridx2[ph, pl.ds(q * 16, 16)] = (
                        ridx_all[pl.ds(b * B + q * 16, 16)])
                compute(ph)
                start_scatter(ph)
        return 0
    lax.fori_loop(0, NBT, batch, 0)

    wait_scatter(0)
    wait_scatter(1)
    plsc.subcore_barrier()

    pltpu.sync_copy(m_sh.at[pl.ds(row0, ZR)],
                    out_hbm.at[cid, pl.ds(row0, ZR)])
    pltpu.sync_copy(m_sh.at[pl.ds(row0 + ZR, ZR)],
                    out_hbm.at[cid, pl.ds(row0 + ZR, ZR)])
    @pl.when(sid == NS - 1)
    def _():
        pltpu.sync_copy(m_sh.at[pl.ds(9984, 16)],
                        out_hbm.at[cid, pl.ds(9984, 16)])


def _sc_edge(y2, ef3, s4, r3):
    mesh = plsc.VectorSubcoreMesh(core_axis_name="c", subcore_axis_name="s",
                                  num_cores=NC, num_subcores=NS)
    fn = pl.kernel(
        _sc_edge_body,
        out_type=jax.ShapeDtypeStruct((NC, N, DH), jnp.float32),
        mesh=mesh,
        compiler_params=pltpu.CompilerParams(use_tc_tiling_on_sc=False,
                                             needs_layout_passes=False),
        scratch_types=[
            pltpu.VMEM((EPT,), jnp.int32),
            pltpu.VMEM((EPT,), jnp.int32),
            pltpu.VMEM((2, B * 4), jnp.float32),
            pltpu.VMEM((2, B), jnp.int32),
            pltpu.VMEM((2, B, RW), jnp.bfloat16),
            pltpu.VMEM((2, B, DH), jnp.float32),
            pltpu.VMEM((ZR, DH), jnp.float32),
            pltpu.VMEM_SHARED((N, DH), jnp.float32),
            pltpu.SemaphoreType.DMA,
            pltpu.SemaphoreType.DMA,
            pltpu.SemaphoreType.DMA,
            pltpu.SemaphoreType.DMA,
            pltpu.SemaphoreType.DMA,
            pltpu.SemaphoreType.DMA,
        ],
    )
    return fn(y2, ef3, s4, r3)


def _skip_body(mp_ref, attr_ref, ws_ref, wl2_ref, out_ref):
    m = jnp.concatenate([mp_ref[0], mp_ref[1]], axis=1)
    wl2 = wl2_ref[...]
    attrs = attr_ref[...]
    acc = m
    for k in range(D_ATTR):
        vk = jnp.dot(ws_ref[k], wl2, preferred_element_type=jnp.float32) * SCALE
        acc = acc + jnp.dot(m, vk,
                            preferred_element_type=jnp.float32) * attrs[:, k:k + 1]
    out_ref[...] = acc


def _tc_skip(m_parts, node_attrs, ws_perm, w_lin2):
    blk = 1000
    return pl.pallas_call(
        _skip_body,
        grid=(N // blk,),
        in_specs=[
            pl.BlockSpec((NC, blk, DH), lambda i: (0, i, 0)),
            pl.BlockSpec((blk, D_ATTR), lambda i: (i, 0)),
            pl.BlockSpec((D_ATTR, D_OUT, D_OUT), lambda i: (0, 0, 0)),
            pl.BlockSpec((D_OUT, D_OUT), lambda i: (0, 0)),
        ],
        out_specs=pl.BlockSpec((blk, D_OUT), lambda i: (i, 0)),
        out_shape=jax.ShapeDtypeStruct((N, D_OUT), jnp.float32),
    )(m_parts, node_attrs, ws_perm, w_lin2)


@jax.jit
def kernel(node_feats, node_attrs, edge_feats, edge_index,
           W_conv, W_lin1, W_skip, W_lin2):
    senders3 = edge_index[0]
    receivers = edge_index[1]
    ef3 = edge_feats.reshape(E * 4)
    wc_perm = W_conv.transpose(1, 0, 2)
    ws_perm = W_skip.transpose(1, 0, 2)
    wl1_perm = W_lin1[:, _PERM]

    yh = _tc_prep(node_feats, wc_perm, wl1_perm)
    y2 = yh.reshape(NC * N, RW)
    m_parts = _sc_edge(y2, ef3, senders3, receivers)
    return _tc_skip(m_parts, node_attrs, ws_perm, W_lin2)
